# baseline (speedup 1.0000x reference)
import functools
import jax
import jax.numpy as jnp
from jax import lax
from jax.experimental import pallas as pl
from jax.experimental.pallas import tpu as pltpu
from jax.experimental.pallas import tpu_sc as plsc

_MM_TILE = 8192
_K = 12
_CHUNK = 256


def _mm_body(x_ref, w_ref, o_ref):
    x = x_ref[...]
    w = w_ref[...]
    s = jax.lax.dot_general(x, w, (((1,), (1,)), ((), ())),
                            preferred_element_type=jnp.float32)
    o_ref[...] = jnp.abs(s)


def _abs_scores(embedding, W):
    B, emb = embedding.shape
    rep = W.shape[0]
    return pl.pallas_call(
        _mm_body,
        grid=(B // _MM_TILE,),
        in_specs=[
            pl.BlockSpec((_MM_TILE, emb), lambda i: (i, 0)),
            pl.BlockSpec(W.shape, lambda i: (0, 0)),
        ],
        out_specs=pl.BlockSpec((_MM_TILE, rep), lambda i: (i, 0)),
        out_shape=jax.ShapeDtypeStruct((B, rep), jnp.float32),
    )(embedding, W)


def _make_sc_topk(B):
    NW = 32
    rows_per_w = B // NW
    n_chunks = rows_per_w // _CHUNK
    mesh = plsc.VectorSubcoreMesh(core_axis_name="c", subcore_axis_name="s")

    @functools.partial(
        pl.kernel, mesh=mesh,
        out_type=jax.ShapeDtypeStruct((B,), jnp.float32),
        compiler_params=pltpu.CompilerParams(needs_layout_passes=False),
        scratch_types=[
            pltpu.VMEM((_CHUNK, 32), jnp.float32),
            pltpu.VMEM((_CHUNK, 32), jnp.float32),
            pltpu.VMEM((_CHUNK,), jnp.float32),
            pltpu.SemaphoreType.DMA,
            pltpu.SemaphoreType.DMA,
        ],
    )
    def sc_topk(a_hbm, out_hbm, buf0, buf1, obuf, sem0, sem1):
        wid = lax.axis_index("s") * 2 + lax.axis_index("c")
        base = wid * rows_per_w
        lane = lax.iota(jnp.int32, 16)
        keep = lane < _K
        bufs = (buf0, buf1)
        sems = (sem0, sem1)

        handle = pltpu.async_copy(
            a_hbm.at[pl.ds(base, _CHUNK), :], bufs[0], sems[0])
        for ci in range(n_chunks):
            start = base + ci * _CHUNK
            buf = bufs[ci % 2]
            handle.wait()
            if ci + 1 < n_chunks:
                handle = pltpu.async_copy(
                    a_hbm.at[pl.ds(start + _CHUNK, _CHUNK), :],
                    bufs[(ci + 1) % 2], sems[(ci + 1) % 2])

            @plsc.parallel_loop(0, _CHUNK, unroll=16)
            def row_body(r):
                a = buf[r, pl.ds(0, 16)]
                b = buf[r, pl.ds(16, 16)]
                sa, _ = plsc.sort_key_val(a, a, descending=True)
                sb, _ = plsc.sort_key_val(b, b)
                hi = jnp.maximum(sa, sb)
                h, _ = plsc.sort_key_val(hi, hi, descending=True)
                s = jnp.sum(jnp.where(keep, h, 0.0), axis=0)
                plsc.store_scatter(
                    obuf,
                    [jnp.full((16,), r, jnp.int32)],
                    jnp.broadcast_to(s * (1.0 / _K), (16,)),
                    mask=lane == 0,
                )

            pltpu.sync_copy(obuf, out_hbm.at[pl.ds(start, _CHUNK)])

    return sc_topk


def kernel(embedding, W):
    B = embedding.shape[0]
    a = _abs_scores(embedding, W)
    out = _make_sc_topk(B)(a)
    return out.reshape(B, 1)


# speedup vs baseline: 3.1249x; 3.1249x over previous
import jax
import jax.numpy as jnp
from jax.experimental import pallas as pl

_B_TILE = 16384
_K = 12


def _tc_body(x_ref, w_ref, o_ref):
    x = x_ref[...]
    w = w_ref[...]
    s = jax.lax.dot_general(w, x, (((1,), (1,)), ((), ())),
                            preferred_element_type=jnp.float32)
    a = jnp.abs(s)
    tb = a.shape[1]
    bits = jax.lax.bitcast_convert_type(a, jnp.int32)
    sub = jax.lax.broadcasted_iota(jnp.int32, a.shape, 0)
    cur = jax.lax.bitcast_convert_type(
        jnp.bitwise_or(jnp.bitwise_and(bits, ~jnp.int32(31)), sub),
        jnp.float32)
    t = [cur[0:8], cur[8:16], cur[16:24], cur[24:32]]

    def _ce(i, j):
        hi = jnp.maximum(t[i], t[j])
        lo = jnp.minimum(t[i], t[j])
        t[i], t[j] = hi, lo

    _ce(0, 1); _ce(2, 3); _ce(0, 2); _ce(1, 3); _ce(1, 2)
    acc = jnp.zeros((1, tb), jnp.float32)
    for _ in range(_K):
        m = jnp.max(t[0], axis=0, keepdims=True)
        acc = acc + m
        eq = t[0] == m
        t[0] = jnp.where(eq, t[1], t[0])
        t[1] = jnp.where(eq, t[2], t[1])
        t[2] = jnp.where(eq, t[3], t[2])
        t[3] = jnp.where(eq, -0.0, t[3])
    o_ref[...] = acc * (1.0 / _K)


def kernel(embedding, W):
    B, emb = embedding.shape
    out = pl.pallas_call(
        _tc_body,
        grid=(B // _B_TILE,),
        in_specs=[
            pl.BlockSpec((_B_TILE, emb), lambda i: (i, 0)),
            pl.BlockSpec(W.shape, lambda i: (0, 0)),
        ],
        out_specs=pl.BlockSpec((1, _B_TILE), lambda i: (0, i)),
        out_shape=jax.ShapeDtypeStruct((1, B), jnp.float32),
    )(embedding, W)
    return out.reshape(B, 1)
